# baseline (speedup 1.0000x reference)
import functools

import jax
import jax.numpy as jnp
from jax import lax
from jax.experimental import pallas as pl
from jax.experimental.pallas import tpu as pltpu
from jax.experimental.pallas import tpu_sc as plsc

B = 16384
D = 128
NC = 2
NS = 16
NW = NC * NS
B_PER_W = B // NW
CHUNK = 128
NCHUNK = B_PER_W // CHUNK


def _encoder_kernel(uids, aids, bids, utab, atab, btab, out,
                    idx_u, idx_a, idx_b, rows_v, isem, gsem):
    wid = lax.axis_index("s") * NC + lax.axis_index("c")
    base = wid * B_PER_W
    tables = ((uids, idx_u, utab, (0, 1, 2)), (aids, idx_a, atab, (3, 4)),
              (bids, idx_b, btab, (5, 6)))
    idx_copies = [
        pltpu.async_copy(ids_hbm.at[pl.ds(base, B_PER_W)], idx_v, isem)
        for ids_hbm, idx_v, _, _ in tables]
    for c in idx_copies:
        c.wait()
    for _, idx_v, tab_hbm, planes in tables:
        copies = []
        for i in range(NCHUNK):
            copies.append(pltpu.async_copy(
                tab_hbm.at[idx_v.at[pl.ds(i * CHUNK, CHUNK)]],
                rows_v.at[pl.ds(i * CHUNK, CHUNK)], gsem))
        for c in copies:
            c.wait()
        for j in planes:
            pltpu.sync_copy(rows_v, out.at[j, pl.ds(base, B_PER_W)])


def kernel(user_ids, item_a_ids, item_b_ids, user_table, item_a_table,
           item_b_table):
    mesh = plsc.VectorSubcoreMesh(core_axis_name="c", subcore_axis_name="s")
    run = functools.partial(
        pl.kernel,
        mesh=mesh,
        out_type=jax.ShapeDtypeStruct((7, B, D), jnp.float32),
        scratch_types=[
            pltpu.VMEM((B_PER_W,), jnp.int32),
            pltpu.VMEM((B_PER_W,), jnp.int32),
            pltpu.VMEM((B_PER_W,), jnp.int32),
            pltpu.VMEM((B_PER_W, D), jnp.float32),
            pltpu.SemaphoreType.DMA,
            pltpu.SemaphoreType.DMA,
        ],
    )(_encoder_kernel)
    return run(user_ids.astype(jnp.int32), item_a_ids.astype(jnp.int32),
               item_b_ids.astype(jnp.int32), user_table, item_a_table,
               item_b_table)


# speedup vs baseline: 1.0171x; 1.0171x over previous
import functools

import jax
import jax.numpy as jnp
from jax import lax
from jax.experimental import pallas as pl
from jax.experimental.pallas import tpu as pltpu
from jax.experimental.pallas import tpu_sc as plsc

B = 16384
D = 128
NC = 2
NS = 16
NW = NC * NS
B_PER_W = B // NW
CHUNK = 128
NCHUNK = B_PER_W // CHUNK


def _encoder_kernel(uids, aids, bids, utab, atab, btab, out,
                    idx_v, rows_v, gsem):
    wid = lax.axis_index("s") * NC + lax.axis_index("c")
    base = wid * B_PER_W
    tables = ((uids, utab, (0, 1, 2)), (aids, atab, (3, 4)),
              (bids, btab, (5, 6)))
    for ids_hbm, tab_hbm, planes in tables:
        pltpu.sync_copy(ids_hbm.at[pl.ds(base, B_PER_W)], idx_v)
        copies = []
        for i in range(NCHUNK):
            copies.append(pltpu.async_copy(
                tab_hbm.at[idx_v.at[pl.ds(i * CHUNK, CHUNK)]],
                rows_v.at[pl.ds(i * CHUNK, CHUNK)], gsem))
        for c in copies:
            c.wait()
        for j in planes:
            pltpu.sync_copy(rows_v, out.at[j, pl.ds(base, B_PER_W)])


def kernel(user_ids, item_a_ids, item_b_ids, user_table, item_a_table,
           item_b_table):
    mesh = plsc.VectorSubcoreMesh(core_axis_name="c", subcore_axis_name="s")
    run = functools.partial(
        pl.kernel,
        mesh=mesh,
        out_type=jax.ShapeDtypeStruct((7, B, D), jnp.float32),
        scratch_types=[
            pltpu.VMEM((B_PER_W,), jnp.int32),
            pltpu.VMEM((B_PER_W, D), jnp.float32),
            pltpu.SemaphoreType.DMA,
        ],
    )(_encoder_kernel)
    return run(user_ids.astype(jnp.int32), item_a_ids.astype(jnp.int32),
               item_b_ids.astype(jnp.int32), user_table, item_a_table,
               item_b_table)
